# baseline (speedup 1.0000x reference)
import functools
import math

import jax
import jax.numpy as jnp
import numpy as np
from jax import lax
from jax.experimental import pallas as pl
from jax.experimental.pallas import tpu as pltpu
from jax.experimental.pallas import tpu_sc as plsc

D = 64
B = 1024
L = 200

_NC = 2
_NS = 16
_NW = _NC * _NS
_N = B * L
_PER_W = _N // _NW
_SUB = 128
_NSUB = _PER_W // _SUB


def _pos_encoding_np(max_len, d):
    pos = np.arange(max_len)[:, None].astype(np.float32)
    i = np.arange(0, d, 2).astype(np.float32)
    div = np.exp(-math.log(10000.0) * i / float(d))
    pe = np.zeros((max_len, d), dtype=np.float32)
    pe[:, 0::2] = np.sin(pos * div)
    pe[:, 1::2] = np.cos(pos * div)
    return pe


_PE2 = np.tile(_pos_encoding_np(L, D), (2, 1))


def _emb_body(idx_hbm, pe_hbm, table_hbm, out_hbm, idx_v, pe_v, row_v, sem):
    wid = lax.axis_index("s") * _NC + lax.axis_index("c")
    base = wid * _PER_W
    pltpu.sync_copy(idx_hbm.at[wid], idx_v)
    pltpu.sync_copy(pe_hbm, pe_v)

    def step(g, carry):
        pltpu.async_copy(table_hbm.at[idx_v.at[g]], row_v, sem).wait()
        p0 = lax.rem(g * _SUB, L)

        def add_pe(j, c):
            pr = p0 + j
            for v in range(D // 16):
                sl = pl.ds(v * 16, 16)
                plsc.addupdate(row_v.at[j, sl], pe_v[pr, sl])
            return c

        lax.fori_loop(0, _SUB, add_pe, 0)
        pltpu.sync_copy(row_v, out_hbm.at[pl.ds(base + g * _SUB, _SUB)])
        return carry

    lax.fori_loop(0, _NSUB, step, 0)


@jax.jit
def _emb(xi, pe2, table):
    f = pl.kernel(
        _emb_body,
        mesh=plsc.VectorSubcoreMesh(core_axis_name="c", subcore_axis_name="s"),
        out_type=jax.ShapeDtypeStruct((_N, D), jnp.float32),
        scratch_types=[
            pltpu.VMEM((_NSUB, _SUB), jnp.int32),
            pltpu.VMEM((2 * L, D), jnp.float32),
            pltpu.VMEM((_SUB, D), jnp.float32),
            pltpu.SemaphoreType.DMA,
        ],
        compiler_params=pltpu.CompilerParams(use_tc_tiling_on_sc=False),
    )
    return f(xi, pe2, table)


def kernel(x, table):
    xi = x.astype(jnp.int32).reshape(_NW, _NSUB, _SUB)
    out = _emb(xi, jnp.asarray(_PE2), table)
    return out.reshape(B, L, D)


# speedup vs baseline: 1.0640x; 1.0640x over previous
import math

import jax
import jax.numpy as jnp
import numpy as np
from jax import lax
from jax.experimental import pallas as pl
from jax.experimental.pallas import tpu as pltpu
from jax.experimental.pallas import tpu_sc as plsc

D = 64
B = 1024
L = 200

_NC = 2
_NS = 16
_NW = _NC * _NS
_N = B * L
_PER_W = _N // _NW
_SUB = 128
_NSUB = _PER_W // _SUB


def _pos_encoding_np(max_len, d):
    pos = np.arange(max_len)[:, None].astype(np.float32)
    i = np.arange(0, d, 2).astype(np.float32)
    div = np.exp(-math.log(10000.0) * i / float(d))
    pe = np.zeros((max_len, d), dtype=np.float32)
    pe[:, 0::2] = np.sin(pos * div)
    pe[:, 1::2] = np.cos(pos * div)
    return pe


_PE2 = np.tile(_pos_encoding_np(L, D), (2, 1))


def _emb_body(idx_hbm, pe_hbm, table_hbm, out_hbm,
              idx_v, pe_v, row0, row1, gsem0, gsem1, ssem0, ssem1):
    wid = lax.axis_index("s") * _NC + lax.axis_index("c")
    base = wid * _PER_W
    pltpu.sync_copy(idx_hbm.at[wid], idx_v)
    pltpu.sync_copy(pe_hbm, pe_v)

    bufs = ((row0, gsem0, ssem0), (row1, gsem1, ssem1))

    def start_gather(g, row, gsem):
        pltpu.async_copy(table_hbm.at[idx_v.at[g]], row, gsem)

    def wait_gather(row, gsem):
        pltpu.make_async_copy(table_hbm.at[pl.ds(0, _SUB)], row, gsem).wait()

    def start_store(g, row, ssem):
        pltpu.async_copy(row, out_hbm.at[pl.ds(base + g * _SUB, _SUB)], ssem)

    def wait_store(row, ssem):
        pltpu.make_async_copy(table_hbm.at[pl.ds(0, _SUB)], row, ssem).wait()

    def add_pe(row, g):
        p0 = lax.rem(g * _SUB, L)

        @pl.loop(0, _SUB, unroll=8)
        def _add(j):
            pr = p0 + j
            for v in range(D // 16):
                sl = pl.ds(v * 16, 16)
                plsc.addupdate(row.at[j, sl], pe_v[pr, sl])

    start_gather(0, row0, gsem0)

    @pl.loop(0, _NSUB // 2)
    def _step(h):
        for b in range(2):
            row, gsem, ssem = bufs[b]
            orow, ogsem, ossem = bufs[1 - b]
            g = h * 2 + b
            wait_gather(row, gsem)
            if b == 0:
                @pl.when(h >= 1)
                def _w():
                    wait_store(orow, ossem)

                start_gather(g + 1, orow, ogsem)
            else:
                wait_store(orow, ossem)

                @pl.when(h < _NSUB // 2 - 1)
                def _g():
                    start_gather(g + 1, orow, ogsem)

            add_pe(row, g)
            start_store(g, row, ssem)

    wait_store(row1, ssem1)


@jax.jit
def _emb(xi, pe2, table):
    f = pl.kernel(
        _emb_body,
        mesh=plsc.VectorSubcoreMesh(core_axis_name="c", subcore_axis_name="s"),
        out_type=jax.ShapeDtypeStruct((_N, D), jnp.float32),
        scratch_types=[
            pltpu.VMEM((_NSUB, _SUB), jnp.int32),
            pltpu.VMEM((2 * L, D), jnp.float32),
            pltpu.VMEM((_SUB, D), jnp.float32),
            pltpu.VMEM((_SUB, D), jnp.float32),
            pltpu.SemaphoreType.DMA,
            pltpu.SemaphoreType.DMA,
            pltpu.SemaphoreType.DMA,
            pltpu.SemaphoreType.DMA,
        ],
        compiler_params=pltpu.CompilerParams(use_tc_tiling_on_sc=False),
    )
    return f(xi, pe2, table)


def kernel(x, table):
    xi = x.astype(jnp.int32).reshape(_NW, _NSUB, _SUB)
    out = _emb(xi, jnp.asarray(_PE2), table)
    return out.reshape(B, L, D)


# speedup vs baseline: 1.0682x; 1.0039x over previous
import math

import jax
import jax.numpy as jnp
import numpy as np
from jax import lax
from jax.experimental import pallas as pl
from jax.experimental.pallas import tpu as pltpu
from jax.experimental.pallas import tpu_sc as plsc

D = 64
B = 1024
L = 200

_NC = 2
_NS = 16
_NW = _NC * _NS
_N = B * L
_PER_W = _N // _NW
_SUB = 128
_NSUB = _PER_W // _SUB


def _pos_encoding_np(max_len, d):
    pos = np.arange(max_len)[:, None].astype(np.float32)
    i = np.arange(0, d, 2).astype(np.float32)
    div = np.exp(-math.log(10000.0) * i / float(d))
    pe = np.zeros((max_len, d), dtype=np.float32)
    pe[:, 0::2] = np.sin(pos * div)
    pe[:, 1::2] = np.cos(pos * div)
    return pe


_PE2 = np.tile(_pos_encoding_np(L, D), (2, 1))


def _emb_body(idx_hbm, pe_hbm, table_hbm, out_hbm,
              idx_v, pe_v, row0, row1, gsem0, gsem1, ssem0, ssem1):
    wid = lax.axis_index("s") * _NC + lax.axis_index("c")
    base = wid * _PER_W
    pltpu.sync_copy(idx_hbm.at[pl.ds(base, _PER_W)], idx_v)
    pltpu.sync_copy(pe_hbm, pe_v)

    bufs = ((row0, gsem0, ssem0), (row1, gsem1, ssem1))

    def start_gather(g, row, gsem):
        pltpu.async_copy(table_hbm.at[idx_v.at[pl.ds(g * _SUB, _SUB)]], row, gsem)

    def wait_gather(row, gsem):
        pltpu.make_async_copy(table_hbm.at[pl.ds(0, _SUB)], row, gsem).wait()

    def start_store(g, row, ssem):
        pltpu.async_copy(row, out_hbm.at[pl.ds(base + g * _SUB, _SUB)], ssem)

    def wait_store(row, ssem):
        pltpu.make_async_copy(table_hbm.at[pl.ds(0, _SUB)], row, ssem).wait()

    def add_pe(row, g):
        p0 = lax.rem(g * _SUB, L)

        @pl.loop(0, _SUB, unroll=8)
        def _add(j):
            pr = p0 + j
            for v in range(D // 16):
                sl = pl.ds(v * 16, 16)
                plsc.addupdate(row.at[j, sl], pe_v[pr, sl])

    start_gather(0, row0, gsem0)

    @pl.loop(0, _NSUB // 2)
    def _step(h):
        for b in range(2):
            row, gsem, ssem = bufs[b]
            orow, ogsem, ossem = bufs[1 - b]
            g = h * 2 + b
            wait_gather(row, gsem)
            if b == 0:
                @pl.when(h >= 1)
                def _w():
                    wait_store(orow, ossem)

                start_gather(g + 1, orow, ogsem)
            else:
                wait_store(orow, ossem)

                @pl.when(h < _NSUB // 2 - 1)
                def _g():
                    start_gather(g + 1, orow, ogsem)

            add_pe(row, g)
            start_store(g, row, ssem)

    wait_store(row1, ssem1)


@jax.jit
def _emb(xi, pe2, table):
    f = pl.kernel(
        _emb_body,
        mesh=plsc.VectorSubcoreMesh(core_axis_name="c", subcore_axis_name="s"),
        out_type=jax.ShapeDtypeStruct((_N, D), jnp.float32),
        scratch_types=[
            pltpu.VMEM((_PER_W,), jnp.int32),
            pltpu.VMEM((2 * L, D), jnp.float32),
            pltpu.VMEM((_SUB, D), jnp.float32),
            pltpu.VMEM((_SUB, D), jnp.float32),
            pltpu.SemaphoreType.DMA,
            pltpu.SemaphoreType.DMA,
            pltpu.SemaphoreType.DMA,
            pltpu.SemaphoreType.DMA,
        ],
        compiler_params=pltpu.CompilerParams(use_tc_tiling_on_sc=False),
    )
    return f(xi, pe2, table)


def kernel(x, table):
    xi = x.astype(jnp.int32).reshape(_N)
    out = _emb(xi, jnp.asarray(_PE2), table)
    return out.reshape(B, L, D)
